# baseline (speedup 1.0000x reference)
import functools

import jax
import jax.numpy as jnp
from jax import lax
from jax.experimental import pallas as pl
from jax.experimental.pallas import tpu as pltpu, tpu_sc as plsc

TOPK = 8
NUM_EXPERTS = 64
NUM_TOKENS = 32768
LANES = 16


def _make_sc_kernel():
    info = plsc.get_sparse_core_info()
    nc, ns = info.num_cores, info.num_subcores
    nw = nc * ns
    assert NUM_TOKENS % nw == 0
    tok_per_w = NUM_TOKENS // nw

    mesh = plsc.VectorSubcoreMesh(core_axis_name="c", subcore_axis_name="s")

    @functools.partial(
        pl.kernel,
        out_type=jax.ShapeDtypeStruct((NUM_TOKENS * 2 * TOPK // 128, 128),
                                      jnp.float32),
        mesh=mesh,
        compiler_params=pltpu.CompilerParams(needs_layout_passes=False,
                                             use_tc_tiling_on_sc=False),
        scratch_types=[
            pltpu.VMEM((tok_per_w, NUM_EXPERTS), jnp.float32),
            pltpu.VMEM((tok_per_w * 2 * TOPK // 128, 128), jnp.float32),
        ],
    )
    def sc_kernel(logits_hbm, out_hbm, lbuf, obuf):
        wid = lax.axis_index("c") * ns + lax.axis_index("s")
        base = wid * tok_per_w

        pltpu.sync_copy(logits_hbm.at[pl.ds(base, tok_per_w)], lbuf)

        iota = lax.iota(jnp.int32, LANES)
        lane_lo = iota < TOPK

        def topk_one(tok):
            ks, vs = [], []
            for g in range(NUM_EXPERTS // LANES):
                x = lbuf[tok, pl.ds(g * LANES, LANES)]
                k_, v_ = plsc.sort_key_val(x, iota + g * LANES,
                                           descending=(g % 2 == 0))
                ks.append(k_)
                vs.append(v_)
            p = jnp.where(lane_lo, ks[0], ks[1])
            pi = jnp.where(lane_lo, vs[0], vs[1])
            q = jnp.where(lane_lo, ks[2], ks[3])
            qi = jnp.where(lane_lo, vs[2], vs[3])
            p, pi = plsc.sort_key_val(p, pi, descending=True)
            q, qi = plsc.sort_key_val(q, qi, descending=False)
            r = jnp.where(lane_lo, p, q)
            ri = jnp.where(lane_lo, pi, qi)
            r, ri = plsc.sort_key_val(r, ri, descending=True)
            e = jnp.where(lane_lo, jnp.exp(r), 0.0)
            w = e / jnp.broadcast_to(jnp.sum(e), (LANES,))
            return w, ri

        @plsc.parallel_loop(0, tok_per_w, unroll=8)
        def body(tok):
            w, ri = topk_one(tok)
            rows = jnp.full((LANES,), tok >> 3, jnp.int32)
            cols = iota + (tok & 7) * (2 * TOPK)
            plsc.store_scatter(obuf, [rows, cols], w, mask=lane_lo)
            plsc.store_scatter(obuf, [rows, cols + TOPK],
                              plsc.bitcast(ri, jnp.float32), mask=lane_lo)

        rows_128 = tok_per_w * 2 * TOPK // 128
        pltpu.sync_copy(obuf, out_hbm.at[pl.ds(wid * rows_128, rows_128)])

    return sc_kernel


_SC_KERNEL = _make_sc_kernel()


def kernel(router_logits_fp32, topk_ids, topk_weights):
    packed = _SC_KERNEL(router_logits_fp32).reshape(NUM_TOKENS, 2 * TOPK)
    w = packed[:, :TOPK].astype(topk_weights.dtype)
    ids = lax.bitcast_convert_type(packed[:, TOPK:], jnp.int32)
    ids = ids.astype(topk_ids.dtype)
    return (w, ids)


# speedup vs baseline: 1.5360x; 1.5360x over previous
import functools

import jax
import jax.numpy as jnp
from jax import lax
from jax.experimental import pallas as pl
from jax.experimental.pallas import tpu as pltpu, tpu_sc as plsc

TOPK = 8
NUM_EXPERTS = 64
NUM_TOKENS = 32768
LANES = 16


def _make_sc_kernel():
    info = plsc.get_sparse_core_info()
    nc, ns = info.num_cores, info.num_subcores
    nw = nc * ns
    assert NUM_TOKENS % nw == 0
    tok_per_w = NUM_TOKENS // nw

    mesh = plsc.VectorSubcoreMesh(core_axis_name="c", subcore_axis_name="s")

    @functools.partial(
        pl.kernel,
        out_type=(
            jax.ShapeDtypeStruct((TOPK, NUM_TOKENS), jnp.float32),
            jax.ShapeDtypeStruct((TOPK, NUM_TOKENS), jnp.int32),
        ),
        mesh=mesh,
        compiler_params=pltpu.CompilerParams(needs_layout_passes=False,
                                             use_tc_tiling_on_sc=False),
        scratch_types=[
            pltpu.VMEM((tok_per_w, NUM_EXPERTS), jnp.float32),
            pltpu.VMEM((TOPK, tok_per_w), jnp.float32),
            pltpu.VMEM((TOPK, tok_per_w), jnp.int32),
        ],
    )
    def sc_kernel(logits_hbm, out_w_hbm, out_i_hbm, lbuf, wbuf, ibuf):
        wid = lax.axis_index("c") * ns + lax.axis_index("s")
        base = wid * tok_per_w

        pltpu.sync_copy(logits_hbm.at[pl.ds(base, tok_per_w)], lbuf)

        iota = lax.iota(jnp.int32, LANES)
        lane_lo = iota < TOPK

        def topk_one(tok):
            ks, vs = [], []
            for g in range(NUM_EXPERTS // LANES):
                x = lbuf[tok, pl.ds(g * LANES, LANES)]
                k_, v_ = plsc.sort_key_val(x, iota + g * LANES,
                                           descending=(g % 2 == 0))
                ks.append(k_)
                vs.append(v_)
            p = jnp.where(lane_lo, ks[0], ks[1])
            pi = jnp.where(lane_lo, vs[0], vs[1])
            q = jnp.where(lane_lo, ks[2], ks[3])
            qi = jnp.where(lane_lo, vs[2], vs[3])
            p, pi = plsc.sort_key_val(p, pi, descending=True)
            q, qi = plsc.sort_key_val(q, qi, descending=False)
            r = jnp.where(lane_lo, p, q)
            ri = jnp.where(lane_lo, pi, qi)
            r, ri = plsc.sort_key_val(r, ri, descending=True)
            e = jnp.where(lane_lo, jnp.exp(r), 0.0)
            w = e / jnp.broadcast_to(jnp.sum(e), (LANES,))
            return w, ri

        @plsc.parallel_loop(0, tok_per_w, unroll=8)
        def body(tok):
            w, ri = topk_one(tok)
            cols = jnp.full((LANES,), tok, jnp.int32)
            plsc.store_scatter(wbuf, [iota, cols], w, mask=lane_lo)
            plsc.store_scatter(ibuf, [iota, cols], ri, mask=lane_lo)

        pltpu.sync_copy(wbuf, out_w_hbm.at[:, pl.ds(base, tok_per_w)])
        pltpu.sync_copy(ibuf, out_i_hbm.at[:, pl.ds(base, tok_per_w)])

    return sc_kernel


_SC_KERNEL = _make_sc_kernel()


def kernel(router_logits_fp32, topk_ids, topk_weights):
    w_t, ids_t = _SC_KERNEL(router_logits_fp32)
    return (w_t.T.astype(topk_weights.dtype), ids_t.T.astype(topk_ids.dtype))


# speedup vs baseline: 1.6225x; 1.0563x over previous
import functools

import jax
import jax.numpy as jnp
from jax import lax
from jax.experimental import pallas as pl
from jax.experimental.pallas import tpu as pltpu, tpu_sc as plsc

TOPK = 8
NUM_EXPERTS = 64
NUM_TOKENS = 32768
LANES = 16


def _make_sc_kernel():
    info = plsc.get_sparse_core_info()
    nc, ns = info.num_cores, info.num_subcores
    nw = nc * ns
    assert NUM_TOKENS % nw == 0
    tok_per_w = NUM_TOKENS // nw

    mesh = plsc.VectorSubcoreMesh(core_axis_name="c", subcore_axis_name="s")

    @functools.partial(
        pl.kernel,
        out_type=(
            jax.ShapeDtypeStruct((NUM_TOKENS // 128, TOPK, 128), jnp.float32),
            jax.ShapeDtypeStruct((NUM_TOKENS // 128, TOPK, 128), jnp.int32),
        ),
        mesh=mesh,
        compiler_params=pltpu.CompilerParams(needs_layout_passes=False,
                                             use_tc_tiling_on_sc=False),
        scratch_types=[
            pltpu.VMEM((tok_per_w, NUM_EXPERTS), jnp.float32),
            pltpu.VMEM((tok_per_w // 128, TOPK, 128), jnp.float32),
            pltpu.VMEM((tok_per_w // 128, TOPK, 128), jnp.int32),
        ],
    )
    def sc_kernel(logits_hbm, out_w_hbm, out_i_hbm, lbuf, wbuf, ibuf):
        wid = lax.axis_index("c") * ns + lax.axis_index("s")
        base = wid * tok_per_w

        pltpu.sync_copy(logits_hbm.at[pl.ds(base, tok_per_w)], lbuf)

        iota = lax.iota(jnp.int32, LANES)
        lane_lo = iota < TOPK

        def topk_one(tok):
            ks, vs = [], []
            for g in range(NUM_EXPERTS // LANES):
                x = lbuf[tok, pl.ds(g * LANES, LANES)]
                k_, v_ = plsc.sort_key_val(x, iota + g * LANES,
                                           descending=(g % 2 == 0))
                ks.append(k_)
                vs.append(v_)
            p = jnp.where(lane_lo, ks[0], ks[1])
            pi = jnp.where(lane_lo, vs[0], vs[1])
            q = jnp.where(lane_lo, ks[2], ks[3])
            qi = jnp.where(lane_lo, vs[2], vs[3])
            p, pi = plsc.sort_key_val(p, pi, descending=True)
            q, qi = plsc.sort_key_val(q, qi, descending=False)
            r = jnp.where(lane_lo, p, q)
            ri = jnp.where(lane_lo, pi, qi)
            r, ri = plsc.sort_key_val(r, ri, descending=True)
            e = jnp.where(lane_lo, jnp.exp(r), 0.0)
            w = e / jnp.broadcast_to(jnp.sum(e), (LANES,))
            return w, ri

        @plsc.parallel_loop(0, tok_per_w, unroll=8)
        def body(tok):
            w, ri = topk_one(tok)
            blk = jnp.full((LANES,), tok >> 7, jnp.int32)
            lane = jnp.full((LANES,), tok & 127, jnp.int32)
            plsc.store_scatter(wbuf, [blk, iota, lane], w, mask=lane_lo)
            plsc.store_scatter(ibuf, [blk, iota, lane], ri, mask=lane_lo)

        nblk = tok_per_w // 128
        pltpu.sync_copy(wbuf, out_w_hbm.at[pl.ds(wid * nblk, nblk)])
        pltpu.sync_copy(ibuf, out_i_hbm.at[pl.ds(wid * nblk, nblk)])

    return sc_kernel


_SC_KERNEL = _make_sc_kernel()


def kernel(router_logits_fp32, topk_ids, topk_weights):
    w3, ids3 = _SC_KERNEL(router_logits_fp32)
    w = w3.transpose(0, 2, 1).reshape(NUM_TOKENS, TOPK)
    ids = ids3.transpose(0, 2, 1).reshape(NUM_TOKENS, TOPK)
    return (w.astype(topk_weights.dtype), ids.astype(topk_ids.dtype))


# speedup vs baseline: 1.7692x; 1.0904x over previous
import functools

import jax
import jax.numpy as jnp
from jax import lax
from jax.experimental import pallas as pl
from jax.experimental.pallas import tpu as pltpu, tpu_sc as plsc

TOPK = 8
NUM_EXPERTS = 64
NUM_TOKENS = 32768
LANES = 16


def _make_sc_kernel():
    info = plsc.get_sparse_core_info()
    nc, ns = info.num_cores, info.num_subcores
    nw = nc * ns
    assert NUM_TOKENS % nw == 0
    tok_per_w = NUM_TOKENS // nw

    mesh = plsc.VectorSubcoreMesh(core_axis_name="c", subcore_axis_name="s")

    @functools.partial(
        pl.kernel,
        out_type=(
            jax.ShapeDtypeStruct((NUM_TOKENS // 128, TOPK, 128), jnp.float32),
            jax.ShapeDtypeStruct((NUM_TOKENS // 128, TOPK, 128), jnp.int32),
        ),
        mesh=mesh,
        compiler_params=pltpu.CompilerParams(needs_layout_passes=False,
                                             use_tc_tiling_on_sc=False),
        scratch_types=[
            pltpu.VMEM((tok_per_w, NUM_EXPERTS), jnp.float32),
            pltpu.VMEM((tok_per_w // 128, TOPK, 129), jnp.float32),
            pltpu.VMEM((tok_per_w // 128, TOPK, 129), jnp.int32),
        ],
    )
    def sc_kernel(logits_hbm, out_w_hbm, out_i_hbm, lbuf, wbuf, ibuf):
        wid = lax.axis_index("c") * ns + lax.axis_index("s")
        base = wid * tok_per_w

        pltpu.sync_copy(logits_hbm.at[pl.ds(base, tok_per_w)], lbuf)

        iota = lax.iota(jnp.int32, LANES)
        lane_lo = iota < TOPK

        def topk_one(tok):
            ks, vs = [], []
            for g in range(NUM_EXPERTS // LANES):
                x = lbuf[tok, pl.ds(g * LANES, LANES)]
                k_, v_ = plsc.sort_key_val(x, iota + g * LANES,
                                           descending=(g % 2 == 0))
                ks.append(k_)
                vs.append(v_)
            p = jnp.where(lane_lo, ks[0], ks[1])
            pi = jnp.where(lane_lo, vs[0], vs[1])
            q = jnp.where(lane_lo, ks[2], ks[3])
            qi = jnp.where(lane_lo, vs[2], vs[3])
            p, pi = plsc.sort_key_val(p, pi, descending=True)
            q, qi = plsc.sort_key_val(q, qi, descending=False)
            r = jnp.where(lane_lo, p, q)
            ri = jnp.where(lane_lo, pi, qi)
            r, ri = plsc.sort_key_val(r, ri, descending=True)
            e = jnp.where(lane_lo, jnp.exp(r), 0.0)
            w = e / jnp.broadcast_to(jnp.sum(e), (LANES,))
            return w, ri

        @plsc.parallel_loop(0, tok_per_w, unroll=8)
        def body(tok):
            w, ri = topk_one(tok)
            blk = jnp.full((LANES,), tok >> 7, jnp.int32)
            lane = jnp.full((LANES,), tok & 127, jnp.int32)
            plsc.store_scatter(wbuf, [blk, iota, lane], w, mask=lane_lo)
            plsc.store_scatter(ibuf, [blk, iota, lane], ri, mask=lane_lo)

        nblk = tok_per_w // 128
        pltpu.sync_copy(wbuf.at[:, :, pl.ds(0, 128)],
                        out_w_hbm.at[pl.ds(wid * nblk, nblk)])
        pltpu.sync_copy(ibuf.at[:, :, pl.ds(0, 128)],
                        out_i_hbm.at[pl.ds(wid * nblk, nblk)])

    return sc_kernel


_SC_KERNEL = _make_sc_kernel()


def kernel(router_logits_fp32, topk_ids, topk_weights):
    w3, ids3 = _SC_KERNEL(router_logits_fp32)
    w = w3.transpose(0, 2, 1).reshape(NUM_TOKENS, TOPK)
    ids = ids3.transpose(0, 2, 1).reshape(NUM_TOKENS, TOPK)
    return (w.astype(topk_weights.dtype), ids.astype(topk_ids.dtype))


# speedup vs baseline: 2.0062x; 1.1339x over previous
import functools

import jax
import jax.numpy as jnp
from jax import lax
from jax.experimental import pallas as pl
from jax.experimental.pallas import tpu as pltpu, tpu_sc as plsc

TOPK = 8
NUM_EXPERTS = 64
NUM_TOKENS = 32768
LANES = 16
CHUNK = 512


def _make_sc_kernel():
    info = plsc.get_sparse_core_info()
    nc, ns = info.num_cores, info.num_subcores
    nw = nc * ns
    assert NUM_TOKENS % nw == 0
    tok_per_w = NUM_TOKENS // nw
    n_chunks = tok_per_w // CHUNK
    blk_per_chunk = CHUNK // 128

    mesh = plsc.VectorSubcoreMesh(core_axis_name="c", subcore_axis_name="s")

    @functools.partial(
        pl.kernel,
        out_type=(
            jax.ShapeDtypeStruct((NUM_TOKENS // 128, TOPK, 128), jnp.float32),
            jax.ShapeDtypeStruct((NUM_TOKENS // 128, TOPK, 128), jnp.int32),
        ),
        mesh=mesh,
        compiler_params=pltpu.CompilerParams(needs_layout_passes=False,
                                             use_tc_tiling_on_sc=True),
        scratch_types=[
            pltpu.VMEM((CHUNK, NUM_EXPERTS), jnp.float32),
            pltpu.VMEM((blk_per_chunk, TOPK, 128), jnp.float32),
            pltpu.VMEM((blk_per_chunk, TOPK, 128), jnp.int32),
        ],
    )
    def sc_kernel(logits_hbm, out_w_hbm, out_i_hbm, lbuf, wbuf, ibuf):
        wid = lax.axis_index("c") * ns + lax.axis_index("s")
        base = wid * tok_per_w

        iota = lax.iota(jnp.int32, LANES)
        lane_lo = iota < TOPK

        def topk_one(tok):
            ks, vs = [], []
            for g in range(NUM_EXPERTS // LANES):
                x = lbuf[tok, pl.ds(g * LANES, LANES)]
                k_, v_ = plsc.sort_key_val(x, iota + g * LANES,
                                           descending=(g % 2 == 0))
                ks.append(k_)
                vs.append(v_)
            p = jnp.where(lane_lo, ks[0], ks[1])
            pi = jnp.where(lane_lo, vs[0], vs[1])
            q = jnp.where(lane_lo, ks[2], ks[3])
            qi = jnp.where(lane_lo, vs[2], vs[3])
            p, pi = plsc.sort_key_val(p, pi, descending=True)
            q, qi = plsc.sort_key_val(q, qi, descending=False)
            r = jnp.where(lane_lo, p, q)
            ri = jnp.where(lane_lo, pi, qi)
            r, ri = plsc.sort_key_val(r, ri, descending=True)
            e = jnp.where(lane_lo, jnp.exp(r), 0.0)
            w = e / jnp.broadcast_to(jnp.sum(e), (LANES,))
            return w, ri

        def do_chunk(ch, _):
            tok0 = base + ch * CHUNK
            pltpu.sync_copy(logits_hbm.at[pl.ds(tok0, CHUNK)], lbuf)

            @plsc.parallel_loop(0, CHUNK, unroll=8)
            def body(tok):
                w, ri = topk_one(tok)
                blk = jnp.full((LANES,), tok >> 7, jnp.int32)
                lane = jnp.full((LANES,), tok & 127, jnp.int32)
                plsc.store_scatter(wbuf, [blk, iota, lane], w, mask=lane_lo)
                plsc.store_scatter(ibuf, [blk, iota, lane], ri, mask=lane_lo)

            ob = (base + ch * CHUNK) // 128
            pltpu.sync_copy(wbuf, out_w_hbm.at[pl.ds(ob, blk_per_chunk)])
            pltpu.sync_copy(ibuf, out_i_hbm.at[pl.ds(ob, blk_per_chunk)])
            return _

        lax.fori_loop(0, n_chunks, do_chunk, 0)

    return sc_kernel


_SC_KERNEL = _make_sc_kernel()


def kernel(router_logits_fp32, topk_ids, topk_weights):
    w3, ids3 = _SC_KERNEL(router_logits_fp32)
    w = w3.transpose(0, 2, 1).reshape(NUM_TOKENS, TOPK)
    ids = ids3.transpose(0, 2, 1).reshape(NUM_TOKENS, TOPK)
    return (w.astype(topk_weights.dtype), ids.astype(topk_ids.dtype))


# speedup vs baseline: 2.0908x; 1.0422x over previous
import functools

import jax
import jax.numpy as jnp
from jax import lax
from jax.experimental import pallas as pl
from jax.experimental.pallas import tpu as pltpu, tpu_sc as plsc

TOPK = 8
NUM_EXPERTS = 64
NUM_TOKENS = 32768
LANES = 16
CHUNK = 256


def _make_sc_kernel():
    info = plsc.get_sparse_core_info()
    nc, ns = info.num_cores, info.num_subcores
    nw = nc * ns
    assert NUM_TOKENS % nw == 0
    tok_per_w = NUM_TOKENS // nw
    n_chunks = tok_per_w // CHUNK
    blk_per_chunk = CHUNK // 128

    mesh = plsc.VectorSubcoreMesh(core_axis_name="c", subcore_axis_name="s")

    @functools.partial(
        pl.kernel,
        out_type=(
            jax.ShapeDtypeStruct((NUM_TOKENS // 128, TOPK, 128), jnp.float32),
            jax.ShapeDtypeStruct((NUM_TOKENS // 128, TOPK, 128), jnp.int32),
        ),
        mesh=mesh,
        compiler_params=pltpu.CompilerParams(needs_layout_passes=False,
                                             use_tc_tiling_on_sc=True),
        scratch_types=[
            pltpu.VMEM((2, CHUNK, NUM_EXPERTS), jnp.float32),
            pltpu.VMEM((blk_per_chunk, TOPK, 128), jnp.float32),
            pltpu.VMEM((blk_per_chunk, TOPK, 128), jnp.int32),
            pltpu.SemaphoreType.DMA,
            pltpu.SemaphoreType.DMA,
        ],
    )
    def sc_kernel(logits_hbm, out_w_hbm, out_i_hbm, lbuf2, wbuf, ibuf,
                  sem0, sem1):
        wid = lax.axis_index("c") * ns + lax.axis_index("s")
        base = wid * tok_per_w

        iota = lax.iota(jnp.int32, LANES)
        lane_lo = iota < TOPK

        def topk_one(lbuf, tok):
            ks, vs = [], []
            for g in range(NUM_EXPERTS // LANES):
                x = lbuf[tok, pl.ds(g * LANES, LANES)]
                k_, v_ = plsc.sort_key_val(x, iota + g * LANES,
                                           descending=(g % 2 == 0))
                ks.append(k_)
                vs.append(v_)
            p = jnp.where(lane_lo, ks[0], ks[1])
            pi = jnp.where(lane_lo, vs[0], vs[1])
            q = jnp.where(lane_lo, ks[2], ks[3])
            qi = jnp.where(lane_lo, vs[2], vs[3])
            p, pi = plsc.sort_key_val(p, pi, descending=True)
            q, qi = plsc.sort_key_val(q, qi, descending=False)
            r = jnp.where(lane_lo, p, q)
            ri = jnp.where(lane_lo, pi, qi)
            r, ri = plsc.sort_key_val(r, ri, descending=True)
            e = jnp.where(lane_lo, jnp.exp(r), 0.0)
            w = e / jnp.broadcast_to(jnp.sum(e), (LANES,))
            return w, ri

        sems = (sem0, sem1)
        copies = [None, None]
        copies[0] = pltpu.async_copy(
            logits_hbm.at[pl.ds(base, CHUNK)], lbuf2.at[0], sems[0])
        for ch in range(n_chunks):
            buf = ch & 1
            if ch + 1 < n_chunks:
                copies[1 - buf] = pltpu.async_copy(
                    logits_hbm.at[pl.ds(base + (ch + 1) * CHUNK, CHUNK)],
                    lbuf2.at[1 - buf], sems[1 - buf])
            copies[buf].wait()
            lbuf = lbuf2.at[buf]

            @plsc.parallel_loop(0, CHUNK, unroll=8)
            def body(tok):
                w, ri = topk_one(lbuf, tok)
                blk = jnp.full((LANES,), tok >> 7, jnp.int32)
                lane = jnp.full((LANES,), tok & 127, jnp.int32)
                plsc.store_scatter(wbuf, [blk, iota, lane], w, mask=lane_lo)
                plsc.store_scatter(ibuf, [blk, iota, lane], ri, mask=lane_lo)

            ob = (base + ch * CHUNK) // 128
            pltpu.sync_copy(wbuf, out_w_hbm.at[pl.ds(ob, blk_per_chunk)])
            pltpu.sync_copy(ibuf, out_i_hbm.at[pl.ds(ob, blk_per_chunk)])

    return sc_kernel


_SC_KERNEL = _make_sc_kernel()


def kernel(router_logits_fp32, topk_ids, topk_weights):
    w3, ids3 = _SC_KERNEL(router_logits_fp32)
    w = w3.transpose(0, 2, 1).reshape(NUM_TOKENS, TOPK)
    ids = ids3.transpose(0, 2, 1).reshape(NUM_TOKENS, TOPK)
    return (w.astype(topk_weights.dtype), ids.astype(topk_ids.dtype))


# speedup vs baseline: 2.1358x; 1.0215x over previous
import functools

import jax
import jax.numpy as jnp
from jax import lax
from jax.experimental import pallas as pl
from jax.experimental.pallas import tpu as pltpu, tpu_sc as plsc

TOPK = 8
NUM_EXPERTS = 64
NUM_TOKENS = 32768
LANES = 16
CHUNK = 256


def _make_sc_kernel():
    info = plsc.get_sparse_core_info()
    nc, ns = info.num_cores, info.num_subcores
    nw = nc * ns
    assert NUM_TOKENS % nw == 0
    tok_per_w = NUM_TOKENS // nw
    n_chunks = tok_per_w // CHUNK
    blk_per_chunk = CHUNK // 128

    mesh = plsc.VectorSubcoreMesh(core_axis_name="c", subcore_axis_name="s")

    @functools.partial(
        pl.kernel,
        out_type=(
            jax.ShapeDtypeStruct((NUM_TOKENS // 128, TOPK, 128), jnp.float32),
            jax.ShapeDtypeStruct((NUM_TOKENS // 128, TOPK, 128), jnp.int32),
        ),
        mesh=mesh,
        compiler_params=pltpu.CompilerParams(needs_layout_passes=False,
                                             use_tc_tiling_on_sc=True),
        scratch_types=[
            pltpu.VMEM((2, CHUNK, NUM_EXPERTS), jnp.float32),
            pltpu.VMEM((2, blk_per_chunk, TOPK, 128), jnp.float32),
            pltpu.VMEM((2, blk_per_chunk, TOPK, 128), jnp.int32),
            pltpu.SemaphoreType.DMA,
            pltpu.SemaphoreType.DMA,
            pltpu.SemaphoreType.DMA,
            pltpu.SemaphoreType.DMA,
        ],
    )
    def sc_kernel(logits_hbm, out_w_hbm, out_i_hbm, lbuf2, wbuf2, ibuf2,
                  sem0, sem1, osem0, osem1):
        wid = lax.axis_index("c") * ns + lax.axis_index("s")
        base = wid * tok_per_w

        iota = lax.iota(jnp.int32, LANES)
        lane_lo = iota < TOPK

        def topk_one(lbuf, tok):
            ks, vs = [], []
            for g in range(NUM_EXPERTS // LANES):
                x = lbuf[tok, pl.ds(g * LANES, LANES)]
                k_, v_ = plsc.sort_key_val(x, iota + g * LANES,
                                           descending=(g % 2 == 0))
                ks.append(k_)
                vs.append(v_)
            p = jnp.where(lane_lo, ks[0], ks[1])
            pi = jnp.where(lane_lo, vs[0], vs[1])
            q = jnp.where(lane_lo, ks[2], ks[3])
            qi = jnp.where(lane_lo, vs[2], vs[3])
            p, pi = plsc.sort_key_val(p, pi, descending=True)
            q, qi = plsc.sort_key_val(q, qi, descending=False)
            r = jnp.where(lane_lo, p, q)
            ri = jnp.where(lane_lo, pi, qi)
            r, ri = plsc.sort_key_val(r, ri, descending=True)
            e = jnp.where(lane_lo, jnp.exp(r), 0.0)
            w = e / jnp.broadcast_to(jnp.sum(e), (LANES,))
            return w, ri

        sems = (sem0, sem1)
        osems = (osem0, osem1)
        copies = [None, None]
        out_copies = [None, None]
        copies[0] = pltpu.async_copy(
            logits_hbm.at[pl.ds(base, CHUNK)], lbuf2.at[0], sems[0])
        for ch in range(n_chunks):
            buf = ch & 1
            if ch + 1 < n_chunks:
                copies[1 - buf] = pltpu.async_copy(
                    logits_hbm.at[pl.ds(base + (ch + 1) * CHUNK, CHUNK)],
                    lbuf2.at[1 - buf], sems[1 - buf])
            copies[buf].wait()
            if out_copies[buf] is not None:
                for c in out_copies[buf]:
                    c.wait()
            lbuf = lbuf2.at[buf]
            wbuf = wbuf2.at[buf]
            ibuf = ibuf2.at[buf]

            @plsc.parallel_loop(0, CHUNK, unroll=8)
            def body(tok):
                w, ri = topk_one(lbuf, tok)
                blk = jnp.full((LANES,), tok >> 7, jnp.int32)
                lane = jnp.full((LANES,), tok & 127, jnp.int32)
                plsc.store_scatter(wbuf, [blk, iota, lane], w, mask=lane_lo)
                plsc.store_scatter(ibuf, [blk, iota, lane], ri, mask=lane_lo)

            ob = (base + ch * CHUNK) // 128
            out_copies[buf] = (
                pltpu.async_copy(wbuf, out_w_hbm.at[pl.ds(ob, blk_per_chunk)],
                                 osems[buf]),
                pltpu.async_copy(ibuf, out_i_hbm.at[pl.ds(ob, blk_per_chunk)],
                                 osems[buf]),
            )
        for oc in out_copies:
            if oc is not None:
                for c in oc:
                    c.wait()

    return sc_kernel


_SC_KERNEL = _make_sc_kernel()


def kernel(router_logits_fp32, topk_ids, topk_weights):
    w3, ids3 = _SC_KERNEL(router_logits_fp32)
    w = w3.transpose(0, 2, 1).reshape(NUM_TOKENS, TOPK)
    ids = ids3.transpose(0, 2, 1).reshape(NUM_TOKENS, TOPK)
    return (w.astype(topk_weights.dtype), ids.astype(topk_ids.dtype))
